# baseline (speedup 1.0000x reference)
import dataclasses
import functools

import jax
import jax.numpy as jnp
from jax.experimental import pallas as pl
from jax.experimental.pallas import tpu as pltpu
from jax.experimental.pallas import tpu_sc as plsc


_LANES = 16
_SC_BLOCK_ROWS = 128


def _sc_scores(target_embeds, context_embeds):
    batch, dim = target_embeds.shape
    nchunks = dim // _LANES
    mesh = plsc.VectorSubcoreMesh(core_axis_name="c", subcore_axis_name="s")
    cp = pltpu.CompilerParams()
    if "needs_layout_passes" in pltpu.CompilerParams.__dataclass_fields__:
        cp = dataclasses.replace(cp, needs_layout_passes=False)

    @functools.partial(
        pl.kernel,
        mesh=mesh,
        out_type=jax.ShapeDtypeStruct((batch,), jnp.float32),
        compiler_params=cp,
    )
    def scores_kernel(t_hbm, c_hbm, o_hbm):
        def body(t_vmem, c_vmem, o_vmem):
            lane_ids = jax.lax.iota(jnp.int32, _LANES)

            @pl.loop(0, _SC_BLOCK_ROWS, step=_LANES)
            def _(r0):
                res = jnp.zeros((_LANES,), jnp.float32)
                for j in range(_LANES):
                    r = r0 + j
                    acc = (
                        t_vmem.at[r, pl.ds(0, _LANES)][...]
                        * c_vmem.at[r, pl.ds(0, _LANES)][...]
                    )
                    for k in range(1, nchunks):
                        acc = acc + (
                            t_vmem.at[r, pl.ds(k * _LANES, _LANES)][...]
                            * c_vmem.at[r, pl.ds(k * _LANES, _LANES)][...]
                        )
                    s = jnp.sum(acc)
                    res = jnp.where(lane_ids == j, s, res)
                o_vmem.at[pl.ds(r0, _LANES)][...] = 1.0 / (1.0 + jnp.exp(-res))

        pltpu.emit_pipeline(
            body,
            grid=(batch // _SC_BLOCK_ROWS,),
            in_specs=[
                pl.BlockSpec((_SC_BLOCK_ROWS, dim), lambda i: (i, 0)),
                pl.BlockSpec((_SC_BLOCK_ROWS, dim), lambda i: (i, 0)),
            ],
            out_specs=[pl.BlockSpec((_SC_BLOCK_ROWS,), lambda i: (i,))],
            core_axis_name=("c", "s"),
            dimension_semantics=(pltpu.PARALLEL,),
        )(t_hbm, c_hbm, o_hbm)

    return scores_kernel(target_embeds, context_embeds)


def kernel(target_embeds, context_embeds):
    return _sc_scores(target_embeds, context_embeds)


# speedup vs baseline: 1.0971x; 1.0971x over previous
import dataclasses
import functools

import jax
import jax.numpy as jnp
from jax import lax
from jax.experimental import pallas as pl
from jax.experimental.pallas import tpu as pltpu
from jax.experimental.pallas import tpu_sc as plsc


_LANES = 16
_NUM_WORKERS = 32
_CHUNK_ROWS = 128


def _sc_scores(target_embeds, context_embeds):
    batch, dim = target_embeds.shape
    nchunks = dim // _LANES
    rows_per_worker = batch // _NUM_WORKERS
    nsteps = rows_per_worker // _CHUNK_ROWS
    mesh = plsc.VectorSubcoreMesh(core_axis_name="c", subcore_axis_name="s")
    cp = pltpu.CompilerParams()
    if "needs_layout_passes" in pltpu.CompilerParams.__dataclass_fields__:
        cp = dataclasses.replace(cp, needs_layout_passes=False)

    @functools.partial(
        pl.kernel,
        mesh=mesh,
        out_type=jax.ShapeDtypeStruct((batch,), jnp.float32),
        compiler_params=cp,
        scratch_types=[
            pltpu.VMEM((2, _CHUNK_ROWS, dim), jnp.float32),
            pltpu.VMEM((2, _CHUNK_ROWS, dim), jnp.float32),
            pltpu.VMEM((rows_per_worker,), jnp.float32),
            pltpu.SemaphoreType.DMA,
            pltpu.SemaphoreType.DMA,
            pltpu.SemaphoreType.DMA,
        ],
    )
    def scores_kernel(t_hbm, c_hbm, o_hbm, t_buf, c_buf, o_buf, tsem, csem, osem):
        wid = lax.axis_index("s") * 2 + lax.axis_index("c")
        base = wid * rows_per_worker

        def chunk_rows(hbm, step):
            return hbm.at[pl.ds(base + step * _CHUNK_ROWS, _CHUNK_ROWS), :]

        pltpu.async_copy(chunk_rows(t_hbm, 0), t_buf.at[0], tsem)
        pltpu.async_copy(chunk_rows(c_hbm, 0), c_buf.at[0], csem)

        lane_ids = lax.iota(jnp.int32, _LANES)
        bfly = [(lane_ids ^ sh).reshape(_LANES, 1) for sh in (8, 4, 2, 1)]
        gdn = lax.GatherDimensionNumbers(
            offset_dims=(), collapsed_slice_dims=(0,), start_index_map=(0,)
        )

        def lane_sum_bcast(v):
            for idx in bfly:
                v = v + lax.gather(
                    v, idx, gdn, (1,),
                    mode=lax.GatherScatterMode.PROMISE_IN_BOUNDS,
                )
            return v

        for step in range(nsteps):
            buf = step % 2
            if step + 1 < nsteps:
                nxt = (step + 1) % 2
                pltpu.async_copy(chunk_rows(t_hbm, step + 1), t_buf.at[nxt], tsem)
                pltpu.async_copy(chunk_rows(c_hbm, step + 1), c_buf.at[nxt], csem)
            pltpu.make_async_copy(chunk_rows(t_hbm, step), t_buf.at[buf], tsem).wait()
            pltpu.make_async_copy(chunk_rows(c_hbm, step), c_buf.at[buf], csem).wait()

            t_v = t_buf.at[buf]
            c_v = c_buf.at[buf]
            obase = step * _CHUNK_ROWS

            @pl.loop(0, _CHUNK_ROWS, step=_LANES)
            def _(r0):
                res = jnp.zeros((_LANES,), jnp.float32)
                half = nchunks // 2
                for j in range(_LANES):
                    r = r0 + j
                    acc_a = (
                        t_v.at[r, pl.ds(0, _LANES)][...]
                        * c_v.at[r, pl.ds(0, _LANES)][...]
                    )
                    acc_b = (
                        t_v.at[r, pl.ds(half * _LANES, _LANES)][...]
                        * c_v.at[r, pl.ds(half * _LANES, _LANES)][...]
                    )
                    for k in range(1, half):
                        acc_a = acc_a + (
                            t_v.at[r, pl.ds(k * _LANES, _LANES)][...]
                            * c_v.at[r, pl.ds(k * _LANES, _LANES)][...]
                        )
                        acc_b = acc_b + (
                            t_v.at[r, pl.ds((half + k) * _LANES, _LANES)][...]
                            * c_v.at[r, pl.ds((half + k) * _LANES, _LANES)][...]
                        )
                    s = lane_sum_bcast(acc_a + acc_b)
                    res = jnp.where(lane_ids == j, s, res)
                o_buf.at[pl.ds(obase + r0, _LANES)][...] = 1.0 / (
                    1.0 + jnp.exp(-res)
                )

        pltpu.async_copy(
            o_buf, o_hbm.at[pl.ds(base, rows_per_worker)], osem
        ).wait()

    return scores_kernel(target_embeds, context_embeds)


def kernel(target_embeds, context_embeds):
    return _sc_scores(target_embeds, context_embeds)


# speedup vs baseline: 5.6430x; 5.1437x over previous
import jax
import jax.numpy as jnp
from jax.experimental import pallas as pl


_ROWS_PER_STEP = 8192
_GROUP = 128


def _dot_sigmoid_body(t_ref, c_ref, o_ref):
    p = t_ref[...] * c_ref[...]
    ngroups = _ROWS_PER_STEP // _GROUP
    p3 = p.reshape(ngroups, _GROUP, _GROUP)
    rows = []
    for g in range(ngroups):
        pt = p3[g].T
        rows.append(jnp.sum(pt, axis=0))
    o_ref[...] = jax.nn.sigmoid(jnp.stack(rows))


def kernel(target_embeds, context_embeds):
    batch, dim = target_embeds.shape
    nsteps = batch // _ROWS_PER_STEP
    ngroups = _ROWS_PER_STEP // _GROUP
    out2d = pl.pallas_call(
        _dot_sigmoid_body,
        grid=(nsteps,),
        in_specs=[
            pl.BlockSpec((_ROWS_PER_STEP, dim), lambda i: (i, 0)),
            pl.BlockSpec((_ROWS_PER_STEP, dim), lambda i: (i, 0)),
        ],
        out_specs=pl.BlockSpec((ngroups, _GROUP), lambda i: (i, 0)),
        out_shape=jax.ShapeDtypeStruct((batch // _GROUP, _GROUP), jnp.float32),
    )(target_embeds, context_embeds)
    return out2d.reshape(batch)


# speedup vs baseline: 5.6696x; 1.0047x over previous
import jax
import jax.numpy as jnp
from jax.experimental import pallas as pl
from jax.experimental.pallas import tpu as pltpu


_ROWS_PER_STEP = 8192
_GROUP = 128


def _dot_sigmoid_body(t_ref, c_ref, o_ref):
    p = t_ref[...] * c_ref[...]
    ngroups = _ROWS_PER_STEP // _GROUP
    p3 = p.reshape(ngroups, _GROUP, _GROUP)
    rows = []
    for g in range(ngroups):
        pt = p3[g].T
        rows.append(jnp.sum(pt, axis=0))
    o_ref[...] = jax.nn.sigmoid(jnp.stack(rows))


def kernel(target_embeds, context_embeds):
    batch, dim = target_embeds.shape
    nsteps = batch // _ROWS_PER_STEP
    ngroups = _ROWS_PER_STEP // _GROUP
    out2d = pl.pallas_call(
        _dot_sigmoid_body,
        grid=(nsteps,),
        in_specs=[
            pl.BlockSpec((_ROWS_PER_STEP, dim), lambda i: (i, 0)),
            pl.BlockSpec((_ROWS_PER_STEP, dim), lambda i: (i, 0)),
        ],
        out_specs=pl.BlockSpec((ngroups, _GROUP), lambda i: (i, 0)),
        out_shape=jax.ShapeDtypeStruct((batch // _GROUP, _GROUP), jnp.float32),
        compiler_params=pltpu.CompilerParams(dimension_semantics=("parallel",)),
    )(target_embeds, context_embeds)
    return out2d.reshape(batch)
